# Initial kernel scaffold; baseline (speedup 1.0000x reference)
import jax
import jax.numpy as jnp
from jax.experimental import pallas as pl


def kernel(x, edge_index, batch, w1, b1, beta2, w2, b2, wg, bg):
    raise NotImplementedError("write your pallas kernel here")


# speedup vs baseline: 2.3854x; 2.3854x over previous
import functools

import jax
import jax.numpy as jnp
from jax.experimental import pallas as pl
from jax.experimental.pallas import tpu as pltpu

_N = 100000
_NUM_GRAPHS = 512


def _dense1_body(x_ref, w1_ref, b1_ref, h_ref, hn_ref):
    h = jnp.maximum(x_ref[...] @ w1_ref[...] + b1_ref[...], 0.0)
    h_ref[...] = h
    nrm = jnp.sqrt(jnp.sum(h * h, axis=-1, keepdims=True))
    hn_ref[...] = h / jnp.maximum(nrm, 1e-12)


def _dense1(x, w1, b1):
    blk = 2000
    n = x.shape[0]
    grid = n // blk
    return pl.pallas_call(
        _dense1_body,
        grid=(grid,),
        in_specs=[
            pl.BlockSpec((blk, 75), lambda i: (i, 0)),
            pl.BlockSpec((75, 16), lambda i: (0, 0)),
            pl.BlockSpec((16,), lambda i: (0,)),
        ],
        out_specs=[
            pl.BlockSpec((blk, 16), lambda i: (i, 0)),
            pl.BlockSpec((blk, 16), lambda i: (i, 0)),
        ],
        out_shape=[
            jax.ShapeDtypeStruct((n, 16), jnp.float32),
            jax.ShapeDtypeStruct((n, 16), jnp.float32),
        ],
    )(x, w1, b1)


def _prop(h, hn, src, dst, beta):
    alpha = beta * jnp.sum(hn[src] * hn[dst], axis=-1)
    ex = jnp.exp(alpha)
    den = jax.ops.segment_sum(ex, dst, num_segments=_N)
    acc = jax.ops.segment_sum(ex[:, None] * h[src], dst, num_segments=_N)
    return acc / (den[:, None] + 1e-30)


def _normalize(h):
    nrm = jnp.sqrt(jnp.sum(h * h, axis=-1, keepdims=True))
    return h / jnp.maximum(nrm, 1e-12)


def kernel(x, edge_index, batch, w1, b1, beta2, w2, b2, wg, bg):
    h, hn = _dense1(x, w1, b1)
    loop = jnp.arange(_N, dtype=edge_index.dtype)
    src = jnp.concatenate([edge_index[0], loop])
    dst = jnp.concatenate([edge_index[1], loop])
    h = _prop(h, hn, src, dst, jnp.float32(1.0))
    h = _prop(h, _normalize(h), src, dst, beta2)
    h = h @ w2 + b2
    y = jax.ops.segment_sum(h, batch, num_segments=_NUM_GRAPHS)
    return y @ wg + bg


# speedup vs baseline: 15.8356x; 6.6384x over previous
import functools

import jax
import jax.numpy as jnp
from jax import lax
from jax.experimental import pallas as pl
from jax.experimental.pallas import tpu as pltpu
from jax.experimental.pallas import tpu_sc as plsc

_N = 100000
_NPAD = 100096
_NG = 512
_E = 3200000
_ET = _E + _N
_WIN = 128
_NWORK = 32
_ETPAD = ((_ET + _WIN * _NWORK - 1) // (_WIN * _NWORK)) * (_WIN * _NWORK)
_W = _ETPAD // _WIN
_WPW = _W // _NWORK
_RPT = _NPAD // 16
_BLK = 3128
_NBLK = _NPAD // _BLK


def _dense1_body(x_ref, w1_ref, b1_ref, ts_ref, td_ref):
    h = jnp.maximum(x_ref[...] @ w1_ref[...] + b1_ref[...][None, :], 0.0)
    nrm = jnp.sqrt(jnp.sum(h * h, axis=-1, keepdims=True))
    hn = h / jnp.maximum(nrm, 1e-12)
    ts_ref[...] = jnp.concatenate([hn, h], axis=1)
    td_ref[...] = hn


def _dense1(x_pad, w1, b1):
    return pl.pallas_call(
        _dense1_body,
        grid=(_NBLK,),
        in_specs=[
            pl.BlockSpec((_BLK, 75), lambda i: (i, 0)),
            pl.BlockSpec((75, 16), lambda i: (0, 0)),
            pl.BlockSpec((16,), lambda i: (0,)),
        ],
        out_specs=[
            pl.BlockSpec((_BLK, 32), lambda i: (i, 0)),
            pl.BlockSpec((_BLK, 16), lambda i: (i, 0)),
        ],
        out_shape=[
            jax.ShapeDtypeStruct((_NPAD, 32), jnp.float32),
            jax.ShapeDtypeStruct((_NPAD, 16), jnp.float32),
        ],
    )(x_pad, w1, b1)


def _sc_prop_body(sidx_h, didx_h, tsrc_h, tdst_h, z16_h, z1_h, acc_o, den_o,
                  sidx_v, didx_v, srows, drows, upd, exv, sem1, sem2,
                  accS, denS):
    c = lax.axis_index("c")
    s = lax.axis_index("s")
    wid = s * 2 + c
    r0 = s * _RPT
    pltpu.sync_copy(z16_h.at[pl.ds(r0, _RPT)], accS.at[pl.ds(r0, _RPT)])
    pltpu.sync_copy(z1_h.at[pl.ds(r0, _RPT)], denS.at[pl.ds(r0, _RPT)])
    plsc.subcore_barrier()

    w0 = wid * _WPW

    def win(i, carry):
        w = w0 + i
        pltpu.sync_copy(sidx_h.at[w], sidx_v)
        pltpu.sync_copy(didx_h.at[w], didx_v)
        pltpu.async_copy(tsrc_h.at[sidx_v], srows, sem1).wait()
        pltpu.async_copy(tdst_h.at[didx_v], drows, sem2).wait()
        for g in range(_WIN // 16):
            rows = lax.iota(jnp.int32, 16) + g * 16
            a = jnp.zeros((16,), jnp.float32)
            for d in range(16):
                col = jnp.full((16,), d, jnp.int32)
                sv = plsc.load_gather(srows, [rows, col])
                dv = plsc.load_gather(drows, [rows, col])
                a = a + sv * dv
            ex = jnp.float32(1.0 / 3628800.0)
            for fct in (362880.0, 40320.0, 5040.0, 720.0, 120.0, 24.0, 6.0,
                        2.0, 1.0, 1.0):
                ex = ex * a + jnp.float32(1.0 / fct)
            exv[pl.ds(g * 16, 16)] = ex
            for d in range(16):
                colh = jnp.full((16,), 16 + d, jnp.int32)
                col = jnp.full((16,), d, jnp.int32)
                hv = plsc.load_gather(srows, [rows, colh])
                plsc.store_scatter(upd, [rows, col], hv * ex)
        pltpu.sync_copy(upd, accS.at[didx_v], add=True)
        pltpu.sync_copy(exv, denS.at[didx_v], add=True)
        return carry

    lax.fori_loop(0, _WPW, win, 0)
    plsc.subcore_barrier()
    pltpu.sync_copy(accS.at[pl.ds(r0, _RPT)], acc_o.at[c, pl.ds(r0, _RPT)])
    pltpu.sync_copy(denS.at[pl.ds(r0, _RPT)], den_o.at[c, pl.ds(r0, _RPT)])


def _sc_prop(sidx, didx, tsrc, tdst, z16, z1):
    mesh = plsc.VectorSubcoreMesh(core_axis_name="c", subcore_axis_name="s",
                                  num_cores=2, num_subcores=16)
    f = pl.kernel(
        _sc_prop_body,
        out_type=[jax.ShapeDtypeStruct((2, _NPAD, 16), jnp.float32),
                  jax.ShapeDtypeStruct((2, _NPAD), jnp.float32)],
        mesh=mesh,
        scratch_types=[
            pltpu.VMEM((_WIN,), jnp.int32),
            pltpu.VMEM((_WIN,), jnp.int32),
            pltpu.VMEM((_WIN, 32), jnp.float32),
            pltpu.VMEM((_WIN, 16), jnp.float32),
            pltpu.VMEM((_WIN, 16), jnp.float32),
            pltpu.VMEM((_WIN,), jnp.float32),
            pltpu.SemaphoreType.DMA,
            pltpu.SemaphoreType.DMA,
            pltpu.VMEM_SHARED((_NPAD, 16), jnp.float32),
            pltpu.VMEM_SHARED((_NPAD,), jnp.float32),
        ],
        compiler_params=pltpu.CompilerParams(needs_layout_passes=False,
                                             use_tc_tiling_on_sc=False),
    )
    return f(sidx, didx, tsrc, tdst, z16, z1)


def _mid_body(acc_ref, den_ref, beta_ref, ts_ref, td_ref):
    den = den_ref[:, 0:1] + den_ref[:, 1:2]
    out = (acc_ref[0] + acc_ref[1]) / (den + 1e-30)
    nrm = jnp.sqrt(jnp.sum(out * out, axis=-1, keepdims=True))
    hn = out / jnp.maximum(nrm, 1e-12)
    ts_ref[...] = jnp.concatenate([hn, out], axis=1)
    td_ref[...] = beta_ref[0, 0] * hn


def _mid(acc, den_t, beta2):
    return pl.pallas_call(
        _mid_body,
        grid=(_NBLK,),
        in_specs=[
            pl.BlockSpec((2, _BLK, 16), lambda i: (0, i, 0)),
            pl.BlockSpec((_BLK, 2), lambda i: (i, 0)),
            pl.BlockSpec((1, 1), lambda i: (0, 0)),
        ],
        out_specs=[
            pl.BlockSpec((_BLK, 32), lambda i: (i, 0)),
            pl.BlockSpec((_BLK, 16), lambda i: (i, 0)),
        ],
        out_shape=[
            jax.ShapeDtypeStruct((_NPAD, 32), jnp.float32),
            jax.ShapeDtypeStruct((_NPAD, 16), jnp.float32),
        ],
    )(acc, den_t, beta2.reshape(1, 1))


def _post_body(acc_ref, den_ref, batch_ref, w2_ref, b2_ref, wg_ref, bg_ref,
               z_ref, y_ref):
    i = pl.program_id(0)

    @pl.when(i == 0)
    def _init():
        y_ref[...] = jnp.zeros_like(y_ref)

    den = den_ref[:, 0:1] + den_ref[:, 1:2]
    out = (acc_ref[0] + acc_ref[1]) / (den + 1e-30)
    h2 = out @ w2_ref[...] + b2_ref[...][None, :]
    ids = batch_ref[0, 0, :]
    oh = (ids[:, None] == lax.broadcasted_iota(jnp.int32, (1, _NG), 1)).astype(jnp.float32)
    h2_hi = h2.astype(jnp.bfloat16).astype(jnp.float32)
    h2_lo = h2 - h2_hi
    y_ref[...] += (jnp.dot(oh.T, h2_hi, preferred_element_type=jnp.float32)
                   + jnp.dot(oh.T, h2_lo, preferred_element_type=jnp.float32))

    @pl.when(i == _NBLK - 1)
    def _fin():
        z_ref[...] = (jnp.dot(y_ref[...], wg_ref[...],
                              preferred_element_type=jnp.float32)
                      + bg_ref[...][None, :])


def _post(acc, den_t, batch3, w2, b2, wg, bg):
    return pl.pallas_call(
        _post_body,
        grid=(_NBLK,),
        in_specs=[
            pl.BlockSpec((2, _BLK, 16), lambda i: (0, i, 0)),
            pl.BlockSpec((_BLK, 2), lambda i: (i, 0)),
            pl.BlockSpec((1, 1, _BLK), lambda i: (i, 0, 0)),
            pl.BlockSpec((16, 64), lambda i: (0, 0)),
            pl.BlockSpec((64,), lambda i: (0,)),
            pl.BlockSpec((64, 1), lambda i: (0, 0)),
            pl.BlockSpec((1,), lambda i: (0,)),
        ],
        out_specs=pl.BlockSpec((_NG, 1), lambda i: (0, 0)),
        out_shape=jax.ShapeDtypeStruct((_NG, 1), jnp.float32),
        scratch_shapes=[pltpu.VMEM((_NG, 64), jnp.float32)],
    )(acc, den_t, batch3, w2, b2, wg, bg)


def _xla_prop(sidx, didx, tsrc, tdst, z16, z1):
    src = sidx.reshape(-1)
    dst = didx.reshape(-1)
    hn_s = tsrc[:, :16]
    h_s = tsrc[:, 16:]
    a = jnp.sum(hn_s[src] * tdst[dst], axis=-1)
    ex = jnp.exp(a)
    den = jax.ops.segment_sum(ex, dst, num_segments=_NPAD)
    acc = jax.ops.segment_sum(ex[:, None] * h_s[src], dst, num_segments=_NPAD)
    accO = jnp.stack([acc, jnp.zeros_like(acc)])
    denO = jnp.stack([den, jnp.zeros_like(den)])
    return accO, denO


def kernel(x, edge_index, batch, w1, b1, beta2, w2, b2, wg, bg):
    x_pad = jnp.pad(x, ((0, _NPAD - _N), (0, 0)))
    loop = jnp.arange(_N, dtype=jnp.int32)
    padi = _N + (jnp.arange(_ETPAD - _ET, dtype=jnp.int32) % (_NPAD - _N))
    src = jnp.concatenate([edge_index[0], loop, padi]).reshape(_W, _WIN)
    dst = jnp.concatenate([edge_index[1], loop, padi]).reshape(_W, _WIN)
    batch3 = jnp.pad(batch, (0, _NPAD - _N), constant_values=_NG + 7).reshape(_NBLK, 1, _BLK)
    z16 = jnp.zeros((_NPAD, 16), jnp.float32)
    z1 = jnp.zeros((_NPAD,), jnp.float32)

    tsrc1, tdst1 = _dense1(x_pad, w1, b1)
    acc1, den1 = _sc_prop(src, dst, tsrc1, tdst1, z16, z1)
    tsrc2, tdst2 = _mid(acc1, den1.T, beta2)
    acc2, den2 = _sc_prop(src, dst, tsrc2, tdst2, z16, z1)
    return _post(acc2, den2.T, batch3, w2, b2, wg, bg)


# speedup vs baseline: 17.0541x; 1.0769x over previous
import functools

import jax
import jax.numpy as jnp
from jax import lax
from jax.experimental import pallas as pl
from jax.experimental.pallas import tpu as pltpu
from jax.experimental.pallas import tpu_sc as plsc

_N = 100000
_NPAD = 100096
_NG = 512
_E = 3200000
_ET = _E + _N
_WIN = 128
_NWORK = 32
_K = 4
_ETPAD = ((_ET + _WIN * _NWORK * _K - 1) // (_WIN * _NWORK * _K)) * (_WIN * _NWORK * _K)
_W = _ETPAD // _WIN
_WPW = _W // _NWORK
_NB = _WPW // _K
_RPT = _NPAD // 16
_BLK = 3128
_NBLK = _NPAD // _BLK


def _dense1_body(x_ref, w1_ref, b1_ref, ts_ref, td_ref):
    h = jnp.maximum(x_ref[...] @ w1_ref[...] + b1_ref[...][None, :], 0.0)
    nrm = jnp.sqrt(jnp.sum(h * h, axis=-1, keepdims=True))
    hn = h / jnp.maximum(nrm, 1e-12)
    ts_ref[...] = jnp.concatenate([hn, h], axis=1)
    td_ref[...] = hn


def _dense1(x_pad, w1, b1):
    return pl.pallas_call(
        _dense1_body,
        grid=(_NBLK,),
        in_specs=[
            pl.BlockSpec((_BLK, 75), lambda i: (i, 0)),
            pl.BlockSpec((75, 16), lambda i: (0, 0)),
            pl.BlockSpec((16,), lambda i: (0,)),
        ],
        out_specs=[
            pl.BlockSpec((_BLK, 32), lambda i: (i, 0)),
            pl.BlockSpec((_BLK, 16), lambda i: (i, 0)),
        ],
        out_shape=[
            jax.ShapeDtypeStruct((_NPAD, 32), jnp.float32),
            jax.ShapeDtypeStruct((_NPAD, 16), jnp.float32),
        ],
    )(x_pad, w1, b1)


def _sc_prop_body(ei_h, tsrc_h, tdst_h, z16_h, z1_h, acc_o, den_o,
                  idxb, srows, drows, upd, exv, asem, dsem,
                  accS, denS):
    c = lax.axis_index("c")
    s = lax.axis_index("s")
    wid = s * 2 + c
    r0 = s * _RPT
    pltpu.sync_copy(z16_h.at[pl.ds(r0, _RPT)], accS.at[pl.ds(r0, _RPT)])
    pltpu.sync_copy(z1_h.at[pl.ds(r0, _RPT)], denS.at[pl.ds(r0, _RPT)])
    plsc.subcore_barrier()

    w0 = wid * _WPW

    def step(j, carry):
        pltpu.sync_copy(ei_h.at[w0 + j], idxb)
        pltpu.sync_copy(tsrc_h.at[idxb.at[0]], srows)
        pltpu.sync_copy(tdst_h.at[idxb.at[1]], drows)
        for g in range(_WIN // 16):
            rows = lax.iota(jnp.int32, 16) + (g * 16)
            a = jnp.zeros((16,), jnp.float32)
            for d in range(16):
                col = jnp.full((16,), d, jnp.int32)
                sv = plsc.load_gather(srows, [rows, col])
                dv = plsc.load_gather(drows, [rows, col])
                a = a + sv * dv
            ex = jnp.float32(1.0 / 3628800.0)
            for fct in (362880.0, 40320.0, 5040.0, 720.0, 120.0, 24.0,
                        6.0, 2.0, 1.0, 1.0):
                ex = ex * a + jnp.float32(1.0 / fct)
            plsc.store_scatter(exv, [rows], ex)
            for d in range(16):
                colh = jnp.full((16,), 16 + d, jnp.int32)
                col = jnp.full((16,), d, jnp.int32)
                hv = plsc.load_gather(srows, [rows, colh])
                plsc.store_scatter(upd, [rows, col], hv * ex)
        pltpu.async_copy(upd, accS.at[idxb.at[1]], asem, add=True)
        pltpu.async_copy(exv, denS.at[idxb.at[1]], dsem, add=True)
        pltpu.make_async_copy(upd, accS.at[idxb.at[1]], asem).wait()
        pltpu.make_async_copy(exv, denS.at[idxb.at[1]], dsem).wait()
        return carry

    lax.fori_loop(0, _WPW, step, 0)
    plsc.subcore_barrier()
    pltpu.sync_copy(accS.at[pl.ds(r0, _RPT)], acc_o.at[c, pl.ds(r0, _RPT)])
    pltpu.sync_copy(denS.at[pl.ds(r0, _RPT)], den_o.at[c, pl.ds(r0, _RPT)])


def _sc_prop(ei, tsrc, tdst, z16, z1):
    mesh = plsc.VectorSubcoreMesh(core_axis_name="c", subcore_axis_name="s",
                                  num_cores=2, num_subcores=16)
    f = pl.kernel(
        _sc_prop_body,
        out_type=[jax.ShapeDtypeStruct((2, _NPAD, 16), jnp.float32),
                  jax.ShapeDtypeStruct((2, _NPAD), jnp.float32)],
        mesh=mesh,
        scratch_types=[
            pltpu.VMEM((2, _WIN), jnp.int32),
            pltpu.VMEM((_WIN, 32), jnp.float32),
            pltpu.VMEM((_WIN, 16), jnp.float32),
            pltpu.VMEM((_WIN, 16), jnp.float32),
            pltpu.VMEM((_WIN,), jnp.float32),
            pltpu.SemaphoreType.DMA,
            pltpu.SemaphoreType.DMA,
            pltpu.VMEM_SHARED((_NPAD, 16), jnp.float32),
            pltpu.VMEM_SHARED((_NPAD,), jnp.float32),
        ],
        compiler_params=pltpu.CompilerParams(needs_layout_passes=False,
                                             use_tc_tiling_on_sc=False),
    )
    return f(ei, tsrc, tdst, z16, z1)


def _mid_body(acc_ref, den_ref, beta_ref, ts_ref, td_ref):
    den = den_ref[:, 0:1] + den_ref[:, 1:2]
    out = (acc_ref[0] + acc_ref[1]) / (den + 1e-30)
    nrm = jnp.sqrt(jnp.sum(out * out, axis=-1, keepdims=True))
    hn = out / jnp.maximum(nrm, 1e-12)
    ts_ref[...] = jnp.concatenate([hn, out], axis=1)
    td_ref[...] = beta_ref[0, 0] * hn


def _mid(acc, den_t, beta2):
    return pl.pallas_call(
        _mid_body,
        grid=(_NBLK,),
        in_specs=[
            pl.BlockSpec((2, _BLK, 16), lambda i: (0, i, 0)),
            pl.BlockSpec((_BLK, 2), lambda i: (i, 0)),
            pl.BlockSpec((1, 1), lambda i: (0, 0)),
        ],
        out_specs=[
            pl.BlockSpec((_BLK, 32), lambda i: (i, 0)),
            pl.BlockSpec((_BLK, 16), lambda i: (i, 0)),
        ],
        out_shape=[
            jax.ShapeDtypeStruct((_NPAD, 32), jnp.float32),
            jax.ShapeDtypeStruct((_NPAD, 16), jnp.float32),
        ],
    )(acc, den_t, beta2.reshape(1, 1))


def _post_body(acc_ref, den_ref, batch_ref, w2_ref, b2_ref, wg_ref, bg_ref,
               z_ref, y_ref):
    i = pl.program_id(0)

    @pl.when(i == 0)
    def _init():
        y_ref[...] = jnp.zeros_like(y_ref)

    den = den_ref[:, 0:1] + den_ref[:, 1:2]
    out = (acc_ref[0] + acc_ref[1]) / (den + 1e-30)
    h2 = out @ w2_ref[...] + b2_ref[...][None, :]
    ids = batch_ref[0, 0, :]
    oh = (ids[:, None] == lax.broadcasted_iota(jnp.int32, (1, _NG), 1)).astype(jnp.float32)
    h2_hi = h2.astype(jnp.bfloat16).astype(jnp.float32)
    h2_lo = h2 - h2_hi
    y_ref[...] += (jnp.dot(oh.T, h2_hi, preferred_element_type=jnp.float32)
                   + jnp.dot(oh.T, h2_lo, preferred_element_type=jnp.float32))

    @pl.when(i == _NBLK - 1)
    def _fin():
        z_ref[...] = (jnp.dot(y_ref[...], wg_ref[...],
                              preferred_element_type=jnp.float32)
                      + bg_ref[...][None, :])


def _post(acc, den_t, batch3, w2, b2, wg, bg):
    return pl.pallas_call(
        _post_body,
        grid=(_NBLK,),
        in_specs=[
            pl.BlockSpec((2, _BLK, 16), lambda i: (0, i, 0)),
            pl.BlockSpec((_BLK, 2), lambda i: (i, 0)),
            pl.BlockSpec((1, 1, _BLK), lambda i: (i, 0, 0)),
            pl.BlockSpec((16, 64), lambda i: (0, 0)),
            pl.BlockSpec((64,), lambda i: (0,)),
            pl.BlockSpec((64, 1), lambda i: (0, 0)),
            pl.BlockSpec((1,), lambda i: (0,)),
        ],
        out_specs=pl.BlockSpec((_NG, 1), lambda i: (0, 0)),
        out_shape=jax.ShapeDtypeStruct((_NG, 1), jnp.float32),
        scratch_shapes=[pltpu.VMEM((_NG, 64), jnp.float32)],
    )(acc, den_t, batch3, w2, b2, wg, bg)


def kernel(x, edge_index, batch, w1, b1, beta2, w2, b2, wg, bg):
    x_pad = jnp.pad(x, ((0, _NPAD - _N), (0, 0)))
    loop = jnp.arange(_N, dtype=jnp.int32)
    padi = _N + (jnp.arange(_ETPAD - _ET, dtype=jnp.int32) % (_NPAD - _N))
    src = jnp.concatenate([edge_index[0], loop, padi]).reshape(_W, 1, _WIN)
    dst = jnp.concatenate([edge_index[1], loop, padi]).reshape(_W, 1, _WIN)
    ei = jnp.concatenate([src, dst], axis=1)
    batch3 = jnp.pad(batch, (0, _NPAD - _N), constant_values=_NG + 7).reshape(_NBLK, 1, _BLK)
    z16 = jnp.zeros((_NPAD, 16), jnp.float32)
    z1 = jnp.zeros((_NPAD,), jnp.float32)

    tsrc1, tdst1 = _dense1(x_pad, w1, b1)
    acc1, den1 = _sc_prop(ei, tsrc1, tdst1, z16, z1)
    tsrc2, tdst2 = _mid(acc1, den1.T, beta2)
    acc2, den2 = _sc_prop(ei, tsrc2, tdst2, z16, z1)
    return _post(acc2, den2.T, batch3, w2, b2, wg, bg)
